# baseline (speedup 1.0000x reference)
import jax
import jax.numpy as jnp
from jax import lax
from jax.experimental import pallas as pl
from jax.experimental.pallas import tpu as pltpu
from jax.experimental.pallas import tpu_sc as plsc

B = 16384
L = 200
D = 64
NW = 32
HB = B // 2
ROWS_PER_W = HB // NW
R = 4
NCHUNK = ROWS_PER_W // R
TOK = R * L
NG = 192 // 16


def _agg_body(ids_hbm, table_hbm, out_hbm,
              ids_a, ids_b, emb_a, emb_b, out_a, out_b, tv,
              gsem_a, gsem_b, osem_a, osem_b):
    wid = lax.axis_index("s") * 2 + lax.axis_index("c")
    row0 = wid * ROWS_PER_W

    ids_v = (ids_a, ids_b)
    emb_v = (emb_a, emb_b)
    out_v = (out_a, out_b)
    gsems = (gsem_a, gsem_b)
    osems = (osem_a, osem_b)

    pltpu.sync_copy(table_hbm.at[pl.ds(0, 4)], tv)

    lane16 = lax.iota(jnp.int32, 16)

    def flat_of(x):
        return ((x & ~(2 * BN - 1)) + ((x & (BN - 1)) << 1)
                + ((x >> 13) & 1))

    def xform_row(b, r, g):
        off = r * L + g * 16
        x = ids_v[b][pl.ds(off, 16)]
        ids_v[b][pl.ds(off, 16)] = flat_of(x)
        return 0

    def issue(c, b):
        base = (row0 + c * R) * L
        pltpu.sync_copy(ids_hbm.at[pl.ds(base, TOK)], ids_v[b])
        for r in range(R):
            xt = ids_v[b][pl.ds(r * L + 184, 16)]
            ids_v[b][pl.ds(r * L + 184, 16)] = jnp.where(
                lane16 >= 8, flat_of(xt), xt)

        def xform(g, _):
            x = ids_v[b][pl.ds(g * 16, 16)]
            ids_v[b][pl.ds(g * 16, 16)] = flat_of(x)
            return 0

        for r in range(R):
            lax.fori_loop(0, NG, lambda g, _, r=r: xform_row(b, r, g), 0)
        for r in range(R):
            pltpu.async_copy(
                table_hbm.at[ids_v[b].at[pl.ds(r * L, 128)]],
                emb_v[b].at[pl.ds(r * L, 128)], gsems[b])
            pltpu.async_copy(
                table_hbm.at[ids_v[b].at[pl.ds(r * L + 128, 72)]],
                emb_v[b].at[pl.ds(r * L + 128, 72)], gsems[b])

    def wait_gathers(b):
        pltpu.make_async_copy(table_hbm.at[pl.ds(0, TOK)], emb_v[b],
                              gsems[b]).wait()

    issue(0, 0)
    issue(1, 1)

    zf = jnp.zeros((16,), jnp.float32)
    lane = lax.iota(jnp.int32, 16)

    @pl.loop(0, NCHUNK, step=2)
    def _chunks(c):
        for b in range(2):
            wait_gathers(b)

            @pl.when(c + b >= 2)
            def _():
                pltpu.make_async_copy(out_v[b], out_hbm.at[pl.ds(row0, R)],
                                      osems[b]).wait()

            for r in range(R):
                def grp(g, carry):
                    a0, a1, a2, a3, c0, c1 = carry
                    off = r * L + g * 16
                    idv = ids_v[b][pl.ds(off, 16)]
                    c0 = c0 + plsc.all_reduce_population_count(idv == 0)
                    c1 = c1 + plsc.all_reduce_population_count(idv == 2)
                    for j in range(16):
                        a0 = a0 + emb_v[b][off + j, pl.ds(0, 16)]
                        a1 = a1 + emb_v[b][off + j, pl.ds(16, 16)]
                        a2 = a2 + emb_v[b][off + j, pl.ds(32, 16)]
                        a3 = a3 + emb_v[b][off + j, pl.ds(48, 16)]
                    return a0, a1, a2, a3, c0, c1

                zi = jnp.zeros((16,), jnp.int32)
                a0, a1, a2, a3, c0, c1 = lax.fori_loop(
                    0, NG, grp, (zf, zf, zf, zf, zi, zi))

                idv = ids_v[b][pl.ds(r * L + 184, 16)]
                tail = lane >= 8
                c0 = c0 + plsc.all_reduce_population_count((idv == 0) & tail)
                c1 = c1 + plsc.all_reduce_population_count((idv == 2) & tail)
                for j in range(8):
                    slot = r * L + 192 + j
                    a0 = a0 + emb_v[b][slot, pl.ds(0, 16)]
                    a1 = a1 + emb_v[b][slot, pl.ds(16, 16)]
                    a2 = a2 + emb_v[b][slot, pl.ds(32, 16)]
                    a3 = a3 + emb_v[b][slot, pl.ds(48, 16)]

                n0 = c0.astype(jnp.float32)
                n1 = c1.astype(jnp.float32)
                inv = jnp.full((16,), jnp.float32(1)) / (
                    jnp.float32(L) - n0 - n1)
                out_v[b][r, pl.ds(0, 16)] = (
                    a0 - n0 * tv[0, pl.ds(0, 16)] - n1 * tv[2, pl.ds(0, 16)]) * inv
                out_v[b][r, pl.ds(16, 16)] = (
                    a1 - n0 * tv[0, pl.ds(16, 16)] - n1 * tv[2, pl.ds(16, 16)]) * inv
                out_v[b][r, pl.ds(32, 16)] = (
                    a2 - n0 * tv[0, pl.ds(32, 16)] - n1 * tv[2, pl.ds(32, 16)]) * inv
                out_v[b][r, pl.ds(48, 16)] = (
                    a3 - n0 * tv[0, pl.ds(48, 16)] - n1 * tv[2, pl.ds(48, 16)]) * inv

            pltpu.async_copy(out_v[b], out_hbm.at[pl.ds(row0 + (c + b) * R, R)],
                             osems[b])

            @pl.when(c + b + 2 < NCHUNK)
            def _():
                issue(c + b + 2, b)

    for b in range(2):
        pltpu.make_async_copy(out_v[b], out_hbm.at[pl.ds(row0, R)],
                              osems[b]).wait()


def _sc_agg(ids_flat, table):
    mesh = plsc.VectorSubcoreMesh(core_axis_name="c", subcore_axis_name="s",
                                  num_cores=2, num_subcores=16)
    fn = pl.kernel(
        _agg_body,
        out_type=jax.ShapeDtypeStruct((HB, D), jnp.float32),
        mesh=mesh,
        compiler_params=pltpu.CompilerParams(needs_layout_passes=False,
                                             use_tc_tiling_on_sc=False),
        scratch_types=[
            pltpu.VMEM((TOK,), jnp.int32),
            pltpu.VMEM((TOK,), jnp.int32),
            pltpu.VMEM((TOK, D), jnp.float32),
            pltpu.VMEM((TOK, D), jnp.float32),
            pltpu.VMEM((R, D), jnp.float32),
            pltpu.VMEM((R, D), jnp.float32),
            pltpu.VMEM((4, D), jnp.float32),
            pltpu.SemaphoreType.DMA,
            pltpu.SemaphoreType.DMA,
            pltpu.SemaphoreType.DMA,
            pltpu.SemaphoreType.DMA,
        ],
    )
    return fn(ids_flat, table)


V = 1000000
BN = 8192
NBLK = (V + 2 * BN - 1) // (2 * BN)
VF = NBLK * 2 * BN


def _relayout_body(in_ref, out_ref):
    x = in_ref[...]
    out_ref[...] = jnp.concatenate([x[:, :BN], x[:, BN:]], axis=0).T


def _relayout(tbT):
    return pl.pallas_call(
        _relayout_body,
        grid=(NBLK,),
        in_specs=[pl.BlockSpec((64, 2 * BN), lambda i: (0, i))],
        out_specs=pl.BlockSpec((BN, 128), lambda i: (i, 0)),
        out_shape=jax.ShapeDtypeStruct((NBLK * BN, 128), jnp.float32),
    )(tbT)


def _fc_body(a_ref, w_ref, b_ref, o_ref):
    o_ref[...] = lax.dot_general(
        a_ref[...], w_ref[...], (((1,), (1,)), ((), ())),
        preferred_element_type=jnp.float32) + b_ref[...]


def _fc(agg, fc_w, fc_b):
    BT = 1024
    return pl.pallas_call(
        _fc_body,
        grid=(HB // BT,),
        in_specs=[
            pl.BlockSpec((BT, D), lambda i: (i, 0)),
            pl.BlockSpec((1024, D), lambda i: (0, 0)),
            pl.BlockSpec((1024,), lambda i: (0,)),
        ],
        out_specs=pl.BlockSpec((BT, 1024), lambda i: (i, 0)),
        out_shape=jax.ShapeDtypeStruct((HB, 1024), jnp.float32),
    )(agg, fc_w, fc_b)


@jax.jit
def kernel(input_ids, emb_table, fc_w, fc_b):
    ids = input_ids.astype(jnp.int32).reshape(-1)
    table = _relayout(emb_table.T).reshape(VF, D)
    agg1 = _sc_agg(ids[:HB * L], table)
    agg2 = _sc_agg(ids[HB * L:], table)
    out1 = _fc(agg1, fc_w, fc_b)
    out2 = _fc(agg2, fc_w, fc_b)
    return jnp.concatenate([out1, out2], axis=0)


# speedup vs baseline: 1.0847x; 1.0847x over previous
import jax
import jax.numpy as jnp
from jax import lax
from jax.experimental import pallas as pl
from jax.experimental.pallas import tpu as pltpu
from jax.experimental.pallas import tpu_sc as plsc

B = 16384
L = 200
D = 64
NW = 32
ROWS_PER_W = B // NW
R = 4
NCHUNK = ROWS_PER_W // R
TOK = R * L
NG = 192 // 16


def _agg_body(ids_hbm, table_hbm, out_hbm,
              ids_a, ids_b, emb_a, emb_b, out_a, out_b, tv,
              gsem_a, gsem_b, osem_a, osem_b):
    wid = lax.axis_index("s") * 2 + lax.axis_index("c")
    row0 = wid * ROWS_PER_W

    ids_v = (ids_a, ids_b)
    emb_v = (emb_a, emb_b)
    out_v = (out_a, out_b)
    gsems = (gsem_a, gsem_b)
    osems = (osem_a, osem_b)

    pltpu.sync_copy(table_hbm.at[pl.ds(0, 4)], tv)

    lane16 = lax.iota(jnp.int32, 16)

    def flat_of(x):
        return ((x & ~(2 * BN - 1)) + ((x & (BN - 1)) << 1)
                + ((x >> 13) & 1))

    def xform_row(b, r, g):
        off = r * L + g * 16
        x = ids_v[b][pl.ds(off, 16)]
        ids_v[b][pl.ds(off, 16)] = flat_of(x)
        return 0

    def issue(c, b):
        base = (row0 + c * R) * L
        pltpu.sync_copy(ids_hbm.at[pl.ds(base, TOK)], ids_v[b])
        for r in range(R):
            xt = ids_v[b][pl.ds(r * L + 184, 16)]
            ids_v[b][pl.ds(r * L + 184, 16)] = jnp.where(
                lane16 >= 8, flat_of(xt), xt)

        def xform(g, _):
            x = ids_v[b][pl.ds(g * 16, 16)]
            ids_v[b][pl.ds(g * 16, 16)] = flat_of(x)
            return 0

        for r in range(R):
            lax.fori_loop(0, NG, lambda g, _, r=r: xform_row(b, r, g), 0)
        for r in range(R):
            pltpu.async_copy(
                table_hbm.at[ids_v[b].at[pl.ds(r * L, 128)]],
                emb_v[b].at[pl.ds(r * L, 128)], gsems[b])
            pltpu.async_copy(
                table_hbm.at[ids_v[b].at[pl.ds(r * L + 128, 72)]],
                emb_v[b].at[pl.ds(r * L + 128, 72)], gsems[b])

    def wait_gathers(b):
        pltpu.make_async_copy(table_hbm.at[pl.ds(0, TOK)], emb_v[b],
                              gsems[b]).wait()

    issue(0, 0)
    issue(1, 1)

    zf = jnp.zeros((16,), jnp.float32)
    lane = lax.iota(jnp.int32, 16)

    @pl.loop(0, NCHUNK, step=2)
    def _chunks(c):
        for b in range(2):
            wait_gathers(b)

            @pl.when(c + b >= 2)
            def _():
                pltpu.make_async_copy(out_v[b], out_hbm.at[pl.ds(row0, R)],
                                      osems[b]).wait()

            for r in range(R):
                def grp(g, carry):
                    a0, a1, a2, a3, c0, c1 = carry
                    off = r * L + g * 16
                    idv = ids_v[b][pl.ds(off, 16)]
                    c0 = c0 + plsc.all_reduce_population_count(idv == 0)
                    c1 = c1 + plsc.all_reduce_population_count(idv == 2)
                    for j in range(16):
                        a0 = a0 + emb_v[b][off + j, pl.ds(0, 16)]
                        a1 = a1 + emb_v[b][off + j, pl.ds(16, 16)]
                        a2 = a2 + emb_v[b][off + j, pl.ds(32, 16)]
                        a3 = a3 + emb_v[b][off + j, pl.ds(48, 16)]
                    return a0, a1, a2, a3, c0, c1

                zi = jnp.zeros((16,), jnp.int32)
                a0, a1, a2, a3, c0, c1 = lax.fori_loop(
                    0, NG, grp, (zf, zf, zf, zf, zi, zi))

                idv = ids_v[b][pl.ds(r * L + 184, 16)]
                tail = lane >= 8
                c0 = c0 + plsc.all_reduce_population_count((idv == 0) & tail)
                c1 = c1 + plsc.all_reduce_population_count((idv == 2) & tail)
                for j in range(8):
                    slot = r * L + 192 + j
                    a0 = a0 + emb_v[b][slot, pl.ds(0, 16)]
                    a1 = a1 + emb_v[b][slot, pl.ds(16, 16)]
                    a2 = a2 + emb_v[b][slot, pl.ds(32, 16)]
                    a3 = a3 + emb_v[b][slot, pl.ds(48, 16)]

                n0 = c0.astype(jnp.float32)
                n1 = c1.astype(jnp.float32)
                inv = jnp.full((16,), jnp.float32(1)) / (
                    jnp.float32(L) - n0 - n1)
                out_v[b][r, pl.ds(0, 16)] = (
                    a0 - n0 * tv[0, pl.ds(0, 16)] - n1 * tv[2, pl.ds(0, 16)]) * inv
                out_v[b][r, pl.ds(16, 16)] = (
                    a1 - n0 * tv[0, pl.ds(16, 16)] - n1 * tv[2, pl.ds(16, 16)]) * inv
                out_v[b][r, pl.ds(32, 16)] = (
                    a2 - n0 * tv[0, pl.ds(32, 16)] - n1 * tv[2, pl.ds(32, 16)]) * inv
                out_v[b][r, pl.ds(48, 16)] = (
                    a3 - n0 * tv[0, pl.ds(48, 16)] - n1 * tv[2, pl.ds(48, 16)]) * inv

            pltpu.async_copy(out_v[b], out_hbm.at[pl.ds(row0 + (c + b) * R, R)],
                             osems[b])

            @pl.when(c + b + 2 < NCHUNK)
            def _():
                issue(c + b + 2, b)

    for b in range(2):
        pltpu.make_async_copy(out_v[b], out_hbm.at[pl.ds(row0, R)],
                              osems[b]).wait()


def _sc_agg(ids_flat, table):
    mesh = plsc.VectorSubcoreMesh(core_axis_name="c", subcore_axis_name="s",
                                  num_cores=2, num_subcores=16)
    fn = pl.kernel(
        _agg_body,
        out_type=jax.ShapeDtypeStruct((B, D), jnp.float32),
        mesh=mesh,
        compiler_params=pltpu.CompilerParams(needs_layout_passes=False,
                                             use_tc_tiling_on_sc=False),
        scratch_types=[
            pltpu.VMEM((TOK,), jnp.int32),
            pltpu.VMEM((TOK,), jnp.int32),
            pltpu.VMEM((TOK, D), jnp.float32),
            pltpu.VMEM((TOK, D), jnp.float32),
            pltpu.VMEM((R, D), jnp.float32),
            pltpu.VMEM((R, D), jnp.float32),
            pltpu.VMEM((4, D), jnp.float32),
            pltpu.SemaphoreType.DMA,
            pltpu.SemaphoreType.DMA,
            pltpu.SemaphoreType.DMA,
            pltpu.SemaphoreType.DMA,
        ],
    )
    return fn(ids_flat, table)


V = 1000000
BN = 8192
NBLK = (V + 2 * BN - 1) // (2 * BN)
VF = NBLK * 2 * BN


def _relayout_body(in_ref, out_ref):
    x = in_ref[...]
    out_ref[...] = jnp.concatenate([x[:, :BN], x[:, BN:]], axis=0).T


def _relayout(tbT):
    return pl.pallas_call(
        _relayout_body,
        grid=(NBLK,),
        in_specs=[pl.BlockSpec((64, 2 * BN), lambda i: (0, i))],
        out_specs=pl.BlockSpec((BN, 128), lambda i: (i, 0)),
        out_shape=jax.ShapeDtypeStruct((NBLK * BN, 128), jnp.float32),
    )(tbT)


def _fc_body(a_ref, w_ref, b_ref, o_ref):
    o_ref[...] = lax.dot_general(
        a_ref[...], w_ref[...], (((1,), (1,)), ((), ())),
        preferred_element_type=jnp.float32) + b_ref[...]


def _fc(agg, fc_w, fc_b):
    BT = 1024
    return pl.pallas_call(
        _fc_body,
        grid=(B // BT,),
        in_specs=[
            pl.BlockSpec((BT, D), lambda i: (i, 0)),
            pl.BlockSpec((1024, D), lambda i: (0, 0)),
            pl.BlockSpec((1024,), lambda i: (0,)),
        ],
        out_specs=pl.BlockSpec((BT, 1024), lambda i: (i, 0)),
        out_shape=jax.ShapeDtypeStruct((B, 1024), jnp.float32),
    )(agg, fc_w, fc_b)


@jax.jit
def kernel(input_ids, emb_table, fc_w, fc_b):
    ids = input_ids.astype(jnp.int32).reshape(-1)
    table = _relayout(emb_table.T).reshape(VF, D)
    agg = _sc_agg(ids, table)
    return _fc(agg, fc_w, fc_b)
